# baseline (speedup 1.0000x reference)
import functools

import jax
import jax.numpy as jnp
from jax import lax
from jax.experimental import pallas as pl
from jax.experimental.pallas import tpu as pltpu
from jax.experimental.pallas import tpu_sc as plsc

N = 10000
E = 320000
D_IN = 128
D_HID = 256
D_OUT = 47

NC = 2
NS = 16
NW = NC * NS
B = 128
NB = 160
NB0 = 160
CH = 20
TOTB = NS * NB
NCHMAX = NB0 // CH
E_PAD = TOTB * B
N_PAD = 10240
RPS = N_PAD // NS
NDMP = RPS // B
D1 = 136
D2 = 64
BR = 200


def _make_seg_sum(D):
    mesh = plsc.VectorSubcoreMesh(core_axis_name="c", subcore_axis_name="s")

    @functools.partial(
        pl.kernel,
        out_type=jax.ShapeDtypeStruct((N_PAD, D), jnp.float32),
        mesh=mesh,
        scratch_types=[
            pltpu.VMEM_SHARED((N_PAD, D), jnp.float32),
            pltpu.VMEM((CH, B), jnp.int32),
            pltpu.VMEM((CH, B), jnp.int32),
            pltpu.VMEM((B, D), jnp.float32),
            pltpu.VMEM((B, D), jnp.float32),
            pltpu.VMEM((NDMP + 1, B), jnp.int32),
            pltpu.VMEM((NCHMAX, CH), jnp.int32),
            pltpu.SemaphoreType.DMA,
            pltpu.SemaphoreType.DMA,
        ],
        compiler_params=pltpu.CompilerParams(use_tc_tiling_on_sc=False),
    )
    def seg_sum(table, src_r, dst_r, zeros_tbl, hidx, bidx, out, acc, src_l,
                dst_l, rows0, rows1, hidx_l, bidx_l, sem0, sem1):
        c = lax.axis_index("c")
        s = lax.axis_index("s")
        base = s * RPS

        @pl.when(c == 0)
        def _body():
            pltpu.sync_copy(hidx.at[s], hidx_l)
            pltpu.sync_copy(bidx.at[s], bidx_l)

            pltpu.async_copy(zeros_tbl.at[hidx_l.at[NDMP]], rows0, sem0).wait()
            for k in range(NDMP):
                pltpu.sync_copy(rows0, acc.at[hidx_l.at[k]])
            plsc.subcore_barrier()

            def chunk(ch, carry):
                cps = pltpu.async_copy(src_r.at[bidx_l.at[ch]], src_l, sem0)
                cpd = pltpu.async_copy(dst_r.at[bidx_l.at[ch]], dst_l, sem1)
                cps.wait()
                cpd.wait()
                pltpu.async_copy(table.at[src_l.at[0]], rows0, sem0)

                def pair(g, carry2):
                    b0 = 2 * g
                    pltpu.async_copy(table.at[src_l.at[b0 + 1]], rows1, sem1)
                    pltpu.make_async_copy(
                        table.at[src_l.at[b0]], rows0, sem0).wait()
                    pltpu.sync_copy(rows0, acc.at[dst_l.at[b0]], add=True)

                    @pl.when(g < CH // 2 - 1)
                    def _():
                        pltpu.async_copy(
                            table.at[src_l.at[b0 + 2]], rows0, sem0)

                    pltpu.make_async_copy(
                        table.at[src_l.at[b0 + 1]], rows1, sem1).wait()
                    pltpu.sync_copy(rows1, acc.at[dst_l.at[b0 + 1]], add=True)
                    return carry2

                lax.fori_loop(0, CH // 2, pair, 0)
                return carry

            lax.fori_loop(0, NCHMAX, chunk, 0)
            plsc.subcore_barrier()

            for k in range(NDMP):
                pltpu.sync_copy(acc.at[pl.ds(base + k * B, B)], rows0)
                pltpu.sync_copy(rows0, out.at[hidx_l.at[k]])

    return seg_sum


_seg_sum_cache = {}


def _seg_sum(D):
    if D not in _seg_sum_cache:
        _seg_sum_cache[D] = _make_seg_sum(D)
    return _seg_sum_cache[D]


def _dense1_body(s01, x, w1l, b1, w1r, w2l, b2, w2r, p_o, z_o, ci_o):
    s = s01[...]
    cnt = jnp.sum(s[:, 128:D1], axis=1, keepdims=True)
    ci = 1.0 / jnp.maximum(cnt, 1.0)
    agg = s[:, :128] * ci
    h = agg @ w1l[...] + b1[...] + x[...] @ w1r[...]
    h = jnp.maximum(h, 0.0)
    p_o[...] = h @ w2l[...]
    z_o[...] = h @ w2r[...] + b2[...]
    ci_o[...] = jnp.broadcast_to(ci, (BR, D2))


def _dense2_body(sp, z, ci, out_o):
    o = sp[...] * ci[...] + z[...]
    col = lax.broadcasted_iota(jnp.int32, (BR, D2), 1)
    om = jnp.where(col < D_OUT, o, -1e30)
    m = jnp.max(om, axis=1, keepdims=True)
    lse = m + jnp.log(jnp.sum(jnp.exp(om - m), axis=1, keepdims=True))
    out_o[...] = (o - lse)[:, :D_OUT]


def _dense1_call(s01, x, w1l, b1, w1r, w2l, b2, w2r):
    grid = (N // BR,)
    return pl.pallas_call(
        _dense1_body,
        grid=grid,
        in_specs=[
            pl.BlockSpec((BR, D1), lambda i: (i, 0)),
            pl.BlockSpec((BR, D_IN), lambda i: (i, 0)),
            pl.BlockSpec((D_IN, D_HID), lambda i: (0, 0)),
            pl.BlockSpec((1, D_HID), lambda i: (0, 0)),
            pl.BlockSpec((D_IN, D_HID), lambda i: (0, 0)),
            pl.BlockSpec((D_HID, D2), lambda i: (0, 0)),
            pl.BlockSpec((1, D2), lambda i: (0, 0)),
            pl.BlockSpec((D_HID, D2), lambda i: (0, 0)),
        ],
        out_specs=[
            pl.BlockSpec((BR, D2), lambda i: (i, 0)),
            pl.BlockSpec((BR, D2), lambda i: (i, 0)),
            pl.BlockSpec((BR, D2), lambda i: (i, 0)),
        ],
        out_shape=[
            jax.ShapeDtypeStruct((N, D2), jnp.float32),
            jax.ShapeDtypeStruct((N, D2), jnp.float32),
            jax.ShapeDtypeStruct((N, D2), jnp.float32),
        ],
    )(s01, x, w1l, b1, w1r, w2l, b2, w2r)


def _dense2_call(sp, z, ci):
    grid = (N // BR,)
    return pl.pallas_call(
        _dense2_body,
        grid=grid,
        in_specs=[
            pl.BlockSpec((BR, D2), lambda i: (i, 0)),
            pl.BlockSpec((BR, D2), lambda i: (i, 0)),
            pl.BlockSpec((BR, D2), lambda i: (i, 0)),
        ],
        out_specs=pl.BlockSpec((BR, D_OUT), lambda i: (i, 0)),
        out_shape=jax.ShapeDtypeStruct((N, D_OUT), jnp.float32),
    )(sp, z, ci)


def kernel(x, edge_index, W1l, b1, W1r, W2l, b2, W2r):
    src = edge_index[0].astype(jnp.int32)
    dst = edge_index[1].astype(jnp.int32)
    pad = E_PAD - E
    src_r = jnp.concatenate([src, jnp.zeros((pad,), jnp.int32)]).reshape(
        TOTB, B)
    dst_r = jnp.concatenate(
        [dst, N + (jnp.arange(pad, dtype=jnp.int32) % (N_PAD - N))]).reshape(
        TOTB, B)

    x_pad = jnp.concatenate(
        [x, jnp.ones((N, 1), jnp.float32), jnp.zeros((N, 7), jnp.float32)],
        axis=1)

    w1l = W1l.T
    w1r = W1r.T
    w2l = jnp.zeros((D_HID, D2), jnp.float32).at[:, :D_OUT].set(W2l.T)
    w2r = jnp.zeros((D_HID, D2), jnp.float32).at[:, :D_OUT].set(W2r.T)
    b2p = jnp.zeros((1, D2), jnp.float32).at[0, :D_OUT].set(b2)

    z1 = jnp.zeros((B, D1), jnp.float32)
    z2 = jnp.zeros((B, D2), jnp.float32)

    srange = jnp.arange(NS, dtype=jnp.int32)
    jrange = jnp.arange(B, dtype=jnp.int32)
    dump_idx = (srange[:, None, None] * RPS
                + jnp.arange(NDMP, dtype=jnp.int32)[None, :, None] * B
                + jrange[None, None, :])
    hidx = jnp.concatenate(
        [dump_idx, jnp.broadcast_to(jrange, (NS, 1, B))], axis=1)
    ch_i = jnp.arange(NCHMAX * CH, dtype=jnp.int32).reshape(NCHMAX, CH)
    bidx = srange[:, None, None] * NB0 + ch_i[None]

    s01 = _seg_sum(D1)(x_pad, src_r, dst_r, z1, hidx, bidx)
    p, z, ci = _dense1_call(s01, x, w1l, b1[None, :], w1r, w2l, b2p, w2r)
    sp = _seg_sum(D2)(p, src_r, dst_r, z2, hidx, bidx)
    return _dense2_call(sp, z, ci)


# speedup vs baseline: 1.0936x; 1.0936x over previous
import functools

import jax
import jax.numpy as jnp
from jax import lax
from jax.experimental import pallas as pl
from jax.experimental.pallas import tpu as pltpu
from jax.experimental.pallas import tpu_sc as plsc

N = 10000
E = 320000
D_IN = 128
D_HID = 256
D_OUT = 47

NC = 2
NS = 16
NW = NC * NS
B = 128
NB = 160
NB0 = 120
NB1 = NB - NB0
CH = 20
TOTB = NS * NB
NCHMAX = NB0 // CH
E_PAD = TOTB * B
N_PAD = 10240
RPS = N_PAD // NS
NDMP = RPS // B
D1 = 136
D2 = 64
BR = 200


def _make_seg_sum(D):
    mesh = plsc.VectorSubcoreMesh(core_axis_name="c", subcore_axis_name="s")

    @functools.partial(
        pl.kernel,
        out_type=jax.ShapeDtypeStruct((NC, N_PAD, D), jnp.float32),
        mesh=mesh,
        scratch_types=[
            pltpu.VMEM_SHARED((N_PAD, D), jnp.float32),
            pltpu.VMEM((CH, B), jnp.int32),
            pltpu.VMEM((CH, B), jnp.int32),
            pltpu.VMEM((B, D), jnp.float32),
            pltpu.VMEM((B, D), jnp.float32),
            pltpu.VMEM((NDMP + 1, B), jnp.int32),
            pltpu.VMEM((NCHMAX, CH), jnp.int32),
            pltpu.SemaphoreType.DMA,
            pltpu.SemaphoreType.DMA,
        ],
        compiler_params=pltpu.CompilerParams(use_tc_tiling_on_sc=False),
    )
    def seg_sum(table, src_r, dst_r, zeros_tbl, hidx, bidx, out, acc, src_l,
                dst_l, rows0, rows1, hidx_l, bidx_l, sem0, sem1):
        c = lax.axis_index("c")
        s = lax.axis_index("s")
        base = s * RPS

        pltpu.sync_copy(hidx.at[s], hidx_l)
        pltpu.sync_copy(bidx.at[c, s], bidx_l)

        pltpu.async_copy(zeros_tbl.at[hidx_l.at[NDMP]], rows0, sem0).wait()
        for k in range(NDMP):
            pltpu.sync_copy(rows0, acc.at[hidx_l.at[k]])
        plsc.subcore_barrier()

        nch = jnp.where(c == 0, NB0 // CH, NB1 // CH)

        def chunk(ch, carry):
            cps = pltpu.async_copy(src_r.at[bidx_l.at[ch]], src_l, sem0)
            cpd = pltpu.async_copy(dst_r.at[bidx_l.at[ch]], dst_l, sem1)
            cps.wait()
            cpd.wait()
            pltpu.async_copy(table.at[src_l.at[0]], rows0, sem0)

            def pair(g, carry2):
                b0 = 2 * g
                pltpu.async_copy(table.at[src_l.at[b0 + 1]], rows1, sem1)
                pltpu.make_async_copy(
                    table.at[src_l.at[b0]], rows0, sem0).wait()
                pltpu.sync_copy(rows0, acc.at[dst_l.at[b0]], add=True)

                @pl.when(g < CH // 2 - 1)
                def _():
                    pltpu.async_copy(table.at[src_l.at[b0 + 2]], rows0, sem0)

                pltpu.make_async_copy(
                    table.at[src_l.at[b0 + 1]], rows1, sem1).wait()
                pltpu.sync_copy(rows1, acc.at[dst_l.at[b0 + 1]], add=True)
                return carry2

            lax.fori_loop(0, CH // 2, pair, 0)
            return carry

        lax.fori_loop(0, nch, chunk, 0)
        plsc.subcore_barrier()

        for k in range(NDMP):
            pltpu.sync_copy(acc.at[pl.ds(base + k * B, B)], rows0)
            pltpu.sync_copy(rows0, out.at[c].at[hidx_l.at[k]])

    return seg_sum


_seg_sum_cache = {}


def _seg_sum(D):
    if D not in _seg_sum_cache:
        _seg_sum_cache[D] = _make_seg_sum(D)
    return _seg_sum_cache[D]


def _dense1_body(s01, x, w1l, b1, w1r, w2l, b2, w2r, p_o, z_o, ci_o):
    s = s01[0] + s01[1]
    cnt = jnp.sum(s[:, 128:D1], axis=1, keepdims=True)
    ci = 1.0 / jnp.maximum(cnt, 1.0)
    agg = s[:, :128] * ci
    h = agg @ w1l[...] + b1[...] + x[...] @ w1r[...]
    h = jnp.maximum(h, 0.0)
    p_o[...] = h @ w2l[...]
    z_o[...] = h @ w2r[...] + b2[...]
    ci_o[...] = jnp.broadcast_to(ci, (BR, D2))


def _dense2_body(sp, z, ci, out_o):
    o = (sp[0] + sp[1]) * ci[...] + z[...]
    col = lax.broadcasted_iota(jnp.int32, (BR, D2), 1)
    om = jnp.where(col < D_OUT, o, -1e30)
    m = jnp.max(om, axis=1, keepdims=True)
    lse = m + jnp.log(jnp.sum(jnp.exp(om - m), axis=1, keepdims=True))
    out_o[...] = (o - lse)[:, :D_OUT]


def _dense1_call(s01, x, w1l, b1, w1r, w2l, b2, w2r):
    grid = (N // BR,)
    return pl.pallas_call(
        _dense1_body,
        grid=grid,
        in_specs=[
            pl.BlockSpec((NC, BR, D1), lambda i: (0, i, 0)),
            pl.BlockSpec((BR, D_IN), lambda i: (i, 0)),
            pl.BlockSpec((D_IN, D_HID), lambda i: (0, 0)),
            pl.BlockSpec((1, D_HID), lambda i: (0, 0)),
            pl.BlockSpec((D_IN, D_HID), lambda i: (0, 0)),
            pl.BlockSpec((D_HID, D2), lambda i: (0, 0)),
            pl.BlockSpec((1, D2), lambda i: (0, 0)),
            pl.BlockSpec((D_HID, D2), lambda i: (0, 0)),
        ],
        out_specs=[
            pl.BlockSpec((BR, D2), lambda i: (i, 0)),
            pl.BlockSpec((BR, D2), lambda i: (i, 0)),
            pl.BlockSpec((BR, D2), lambda i: (i, 0)),
        ],
        out_shape=[
            jax.ShapeDtypeStruct((N, D2), jnp.float32),
            jax.ShapeDtypeStruct((N, D2), jnp.float32),
            jax.ShapeDtypeStruct((N, D2), jnp.float32),
        ],
    )(s01, x, w1l, b1, w1r, w2l, b2, w2r)


def _dense2_call(sp, z, ci):
    grid = (N // BR,)
    return pl.pallas_call(
        _dense2_body,
        grid=grid,
        in_specs=[
            pl.BlockSpec((NC, BR, D2), lambda i: (0, i, 0)),
            pl.BlockSpec((BR, D2), lambda i: (i, 0)),
            pl.BlockSpec((BR, D2), lambda i: (i, 0)),
        ],
        out_specs=pl.BlockSpec((BR, D_OUT), lambda i: (i, 0)),
        out_shape=jax.ShapeDtypeStruct((N, D_OUT), jnp.float32),
    )(sp, z, ci)


def kernel(x, edge_index, W1l, b1, W1r, W2l, b2, W2r):
    src = edge_index[0].astype(jnp.int32)
    dst = edge_index[1].astype(jnp.int32)
    pad = E_PAD - E
    src_r = jnp.concatenate([src, jnp.zeros((pad,), jnp.int32)]).reshape(
        TOTB, B)
    dst_r = jnp.concatenate(
        [dst, N + (jnp.arange(pad, dtype=jnp.int32) % (N_PAD - N))]).reshape(
        TOTB, B)

    x_pad = jnp.concatenate(
        [x, jnp.ones((N, 1), jnp.float32), jnp.zeros((N, 7), jnp.float32)],
        axis=1)

    w1l = W1l.T
    w1r = W1r.T
    w2l = jnp.zeros((D_HID, D2), jnp.float32).at[:, :D_OUT].set(W2l.T)
    w2r = jnp.zeros((D_HID, D2), jnp.float32).at[:, :D_OUT].set(W2r.T)
    b2p = jnp.zeros((1, D2), jnp.float32).at[0, :D_OUT].set(b2)

    z1 = jnp.zeros((B, D1), jnp.float32)
    z2 = jnp.zeros((B, D2), jnp.float32)

    srange = jnp.arange(NS, dtype=jnp.int32)
    jrange = jnp.arange(B, dtype=jnp.int32)
    dump_idx = (srange[:, None, None] * RPS
                + jnp.arange(NDMP, dtype=jnp.int32)[None, :, None] * B
                + jrange[None, None, :])
    hidx = jnp.concatenate(
        [dump_idx, jnp.broadcast_to(jrange, (NS, 1, B))], axis=1)
    ch_i = jnp.arange(NCHMAX * CH, dtype=jnp.int32).reshape(NCHMAX, CH)
    b0v = srange[:, None, None] * NB0 + ch_i[None]
    b1v = NS * NB0 + srange[:, None, None] * NB1 + ch_i[None]
    b1v = jnp.where(ch_i[None] < NB1, b1v, 0)
    bidx = jnp.stack([b0v, b1v], axis=0)

    s01 = _seg_sum(D1)(x_pad, src_r, dst_r, z1, hidx, bidx)
    p, z, ci = _dense1_call(s01, x, w1l, b1[None, :], w1r, w2l, b2p, w2r)
    sp = _seg_sum(D2)(p, src_r, dst_r, z2, hidx, bidx)
    return _dense2_call(sp, z, ci)
